# Initial kernel scaffold; baseline (speedup 1.0000x reference)
import jax
import jax.numpy as jnp
from jax.experimental import pallas as pl


def kernel(t_in, y_in, alpha, beta, r1_idx, p1_idx, r2a_idx, r2b_idx, p2_idx):
    raise NotImplementedError("write your pallas kernel here")


# speedup vs baseline: 4.3009x; 4.3009x over previous
import jax
import jax.numpy as jnp
from jax import lax
from jax.experimental import pallas as pl
from jax.experimental.pallas import tpu as pltpu

B = 1024
N = 1024
R1 = 16384
R2 = 16384
T = 1024
NSTEPS = R1 // T


def _body(t_ref, y_ref,
          a1_ref, b1_ref, r1l_ref, r1s_ref, p1s_ref,
          a2_ref, b2_ref, r2al_ref, r2bl_ref, r2as_ref, r2bs_ref, p2s_ref,
          out_ref):
    i = pl.program_id(0)
    bf16 = jnp.bfloat16

    negt = -t_ref[...]
    den = 1.0 + jnp.exp(negt)
    y = y_ref[...]

    iota_s = lax.broadcasted_iota(jnp.int32, (N, T), 0)
    iota_l = lax.broadcasted_iota(jnp.int32, (T, N), 1)

    def dot(a, b):
        return lax.dot_general(a, b, (((1,), (0,)), ((), ())),
                               preferred_element_type=jnp.float32)

    r1l = r1l_ref[0]
    a1 = a1_ref[0]
    b1 = b1_ref[0]
    G1 = (iota_s == r1l).astype(bf16)
    g1 = dot(y, G1)
    c1 = a1 * jnp.exp(b1 * negt)
    rates1 = (c1 * g1).astype(bf16)
    r1s = r1s_ref[0]
    p1s = p1s_ref[0]
    M1 = ((p1s == iota_l).astype(jnp.float32)
          - (r1s == iota_l).astype(jnp.float32)).astype(bf16)
    acc = dot(rates1, M1)

    r2al = r2al_ref[0]
    r2bl = r2bl_ref[0]
    a2 = a2_ref[0]
    b2 = b2_ref[0]
    G2a = (iota_s == r2al).astype(bf16)
    G2b = (iota_s == r2bl).astype(bf16)
    g2a = dot(y, G2a)
    g2b = dot(y, G2b)
    c2 = a2 * jnp.exp(b2 * negt)
    rates2 = (c2 * g2a * g2b * den).astype(bf16)
    r2as = r2as_ref[0]
    r2bs = r2bs_ref[0]
    p2s = p2s_ref[0]
    M2 = ((p2s == iota_l).astype(jnp.float32)
          - (r2as == iota_l).astype(jnp.float32)
          - (r2bs == iota_l).astype(jnp.float32)).astype(bf16)
    acc = acc + dot(rates2, M2)

    @pl.when(i == 0)
    def _():
        out_ref[...] = acc

    @pl.when(i > 0)
    def _():
        out_ref[...] = out_ref[...] + acc


def kernel(t_in, y_in, alpha, beta, r1_idx, p1_idx, r2a_idx, r2b_idx, p2_idx):
    t_col = t_in.reshape(B, 1)
    y_bf = y_in.astype(jnp.bfloat16)

    def lanes(x):
        return x.reshape(NSTEPS, 1, T)

    def subl(x):
        return x.reshape(NSTEPS, T, 1)

    a1, a2 = lanes(alpha[:R1]), lanes(alpha[R1:])
    b1, b2 = lanes(beta[:R1]), lanes(beta[R1:])

    lane_spec = pl.BlockSpec((1, 1, T), lambda i: (i, 0, 0))
    sub_spec = pl.BlockSpec((1, T, 1), lambda i: (i, 0, 0))
    full2d = pl.BlockSpec((B, N), lambda i: (0, 0))

    return pl.pallas_call(
        _body,
        grid=(NSTEPS,),
        in_specs=[
            pl.BlockSpec((B, 1), lambda i: (0, 0)),
            full2d,
            lane_spec, lane_spec,
            lane_spec, sub_spec, sub_spec,
            lane_spec, lane_spec,
            lane_spec, lane_spec,
            sub_spec, sub_spec, sub_spec,
        ],
        out_specs=full2d,
        out_shape=jax.ShapeDtypeStruct((B, N), jnp.float32),
        compiler_params=pltpu.CompilerParams(
            dimension_semantics=("arbitrary",),
        ),
    )(t_col, y_bf,
      a1, b1, lanes(r1_idx), subl(r1_idx), subl(p1_idx),
      a2, b2, lanes(r2a_idx), lanes(r2b_idx),
      subl(r2a_idx), subl(r2b_idx), subl(p2_idx))
